# baseline (speedup 1.0000x reference)
import functools

import jax
import jax.numpy as jnp
from jax import lax
from jax.experimental import pallas as pl
from jax.experimental.pallas import tpu as pltpu
from jax.experimental.pallas import tpu_sc as plsc

_B = 16384
_H = 200
_D = 16
_P = _D + 1
_NC = 2
_NS = 16
_NW = _NC * _NS
_CH = 1024
_NBC = _B // _CH
_UNITS = _H * _NBC
_PER_W = _UNITS // _NW
_HALF = _CH * 8


def _make_gather():
    mesh = plsc.VectorSubcoreMesh(core_axis_name="c", subcore_axis_name="s")

    @functools.partial(
        pl.kernel,
        mesh=mesh,
        out_type=jax.ShapeDtypeStruct((_H, 2, _B * 8), jnp.float32),
        scratch_types=[
            pltpu.VMEM((2, _CH), jnp.int32),
            pltpu.VMEM((2, _CH, _D), jnp.float32),
            pltpu.VMEM((_CH * _P,), jnp.float32),
            pltpu.VMEM((2, 2 * _HALF), jnp.float32),
            pltpu.SemaphoreType.DMA,
            pltpu.SemaphoreType.DMA,
            pltpu.SemaphoreType.DMA,
            pltpu.SemaphoreType.DMA,
        ],
        compiler_params=pltpu.CompilerParams(use_tc_tiling_on_sc=False,
                                             needs_layout_passes=False),
    )
    def k(xt_hbm, w_hbm, out_hbm, idx_v, rows_v, rows_p, t_v,
          sem0, sem1, tsem0, tsem1):
        wid = lax.axis_index("s") * _NC + lax.axis_index("c")
        u0 = wid * _PER_W
        lanes = lax.iota(jnp.int32, 16)
        pos17 = [lanes * _P + d for d in range(16)]
        sems = [sem0, sem1]
        tsems = [tsem0, tsem1]

        def stage(u, buf, sem):
            h = u // _NBC
            bcb = u % _NBC
            pltpu.sync_copy(xt_hbm.at[h, pl.ds(bcb * _CH, _CH)],
                            idx_v.at[buf])
            return pltpu.async_copy(w_hbm.at[idx_v.at[buf]],
                                    rows_v.at[buf], sem)

        stage(u0, 0, sems[0])

        def unit(j, _):
            u = u0 + j
            h = u // _NBC
            bcb = u % _NBC
            for b in range(2):

                @pl.when(j % 2 == b)
                def _():
                    pltpu.make_async_copy(w_hbm.at[idx_v.at[b]],
                                          rows_v.at[b], sems[b]).wait()

                    @pl.when(j + 1 < _PER_W)
                    def _():
                        stage(u + 1, 1 - b, sems[1 - b])

                    @pl.when(j >= 2)
                    def _():
                        for dh in range(2):
                            pltpu.make_async_copy(
                                t_v.at[b, pl.ds(dh * _HALF, _HALF)],
                                out_hbm.at[h, dh, pl.ds(bcb * _HALF, _HALF)],
                                tsems[b]).wait()

                    @plsc.parallel_loop(0, _CH // 16, unroll=16)
                    def repack(r0):
                        rp = rows_p.at[pl.ds(r0 * 16 * _P, 16 * _P)]
                        r = r0 * 16
                        for i in range(16):
                            rp[pl.ds(i * _P, _D)] = rows_v[b, r + i, :]

                    @plsc.parallel_loop(0, _CH // 128, unroll=8)
                    def block(bcl):
                        rp = rows_p.at[pl.ds(bcl * 128 * _P, 128 * _P)]
                        tv = t_v.at[b, pl.ds(bcl * 1024, _HALF + 1024)]
                        for d in range(16):
                            toff = (d // 8) * _HALF + (d % 8) * 128
                            for bi in range(8):
                                v = plsc.load_gather(
                                    rp, [pos17[d] + bi * 16 * _P])
                                tv[pl.ds(toff + bi * 16, 16)] = v

                    for dh in range(2):
                        pltpu.async_copy(
                            t_v.at[b, pl.ds(dh * _HALF, _HALF)],
                            out_hbm.at[h, dh, pl.ds(bcb * _HALF, _HALF)],
                            tsems[b])
            return 0

        lax.fori_loop(0, _PER_W, unit, 0)
        for b in range(2):
            for dh in range(2):
                pltpu.make_async_copy(
                    t_v.at[b, pl.ds(dh * _HALF, _HALF)],
                    out_hbm.at[0, dh, pl.ds(0, _HALF)], tsems[b]).wait()

    return k


def kernel(x, W):
    x_t = jnp.swapaxes(x, 0, 1).astype(jnp.int32)
    out5 = _make_gather()(x_t, W)
    t = out5.reshape(_H, 2, _B // 128, 8, 128)
    t = t.transpose(0, 1, 3, 2, 4)
    t = t.reshape(_H, _D, _B)
    return t.transpose(2, 0, 1)


# speedup vs baseline: 1.1711x; 1.1711x over previous
import functools

import jax
import jax.numpy as jnp
from jax import lax
from jax.experimental import pallas as pl
from jax.experimental.pallas import tpu as pltpu
from jax.experimental.pallas import tpu_sc as plsc

_B = 16384
_H = 200
_D = 16
_P = _D + 1
_NC = 2
_NS = 16
_NW = _NC * _NS
_CH = 1024
_NBC = _B // _CH
_UNITS = _H * _NBC
_PER_W = _UNITS // _NW
_HALF = _CH * 8


def _make_gather():
    mesh = plsc.VectorSubcoreMesh(core_axis_name="c", subcore_axis_name="s")

    @functools.partial(
        pl.kernel,
        mesh=mesh,
        out_type=jax.ShapeDtypeStruct((_H, 2, _B * 8), jnp.float32),
        scratch_types=[
            pltpu.VMEM((2, _CH), jnp.int32),
            pltpu.VMEM((2, _CH, _D), jnp.float32),
            pltpu.VMEM((_CH * _P,), jnp.float32),
            pltpu.VMEM((2, 2 * _HALF), jnp.float32),
            pltpu.SemaphoreType.DMA,
            pltpu.SemaphoreType.DMA,
            pltpu.SemaphoreType.DMA,
            pltpu.SemaphoreType.DMA,
        ],
        compiler_params=pltpu.CompilerParams(use_tc_tiling_on_sc=False,
                                             needs_layout_passes=False),
    )
    def k(xt_hbm, w_hbm, out_hbm, idx_v, rows_v, rows_p, t_v,
          sem0, sem1, tsem0, tsem1):
        wid = lax.axis_index("s") * _NC + lax.axis_index("c")
        u0 = wid * _PER_W
        lanes = lax.iota(jnp.int32, 16)
        pos17 = [lanes * _P + d for d in range(16)]
        sems = [sem0, sem1]
        tsems = [tsem0, tsem1]

        def stage(u, buf, sem):
            h = u // _NBC
            bcb = u % _NBC
            pltpu.sync_copy(xt_hbm.at[h, pl.ds(bcb * _CH, _CH)],
                            idx_v.at[buf])
            return pltpu.async_copy(w_hbm.at[idx_v.at[buf]],
                                    rows_v.at[buf], sem)

        stage(u0, 0, sems[0])

        def unit(j, _):
            u = u0 + j
            h = u // _NBC
            bcb = u % _NBC
            for b in range(2):

                @pl.when(j % 2 == b)
                def _():
                    pltpu.make_async_copy(w_hbm.at[idx_v.at[b]],
                                          rows_v.at[b], sems[b]).wait()

                    @pl.when(j + 1 < _PER_W)
                    def _():
                        stage(u + 1, 1 - b, sems[1 - b])

                    @pl.when(j >= 2)
                    def _():
                        for dh in range(2):
                            pltpu.make_async_copy(
                                t_v.at[b, pl.ds(dh * _HALF, _HALF)],
                                out_hbm.at[h, dh, pl.ds(bcb * _HALF, _HALF)],
                                tsems[b]).wait()

                    @plsc.parallel_loop(0, _CH // 16, unroll=8)
                    def repack(r0):
                        rp = rows_p.at[pl.ds(r0 * 16 * _P, 16 * _P)]
                        r = r0 * 16
                        for i in range(16):
                            rp[pl.ds(i * _P, _D)] = rows_v[b, r + i, :]

                    @plsc.parallel_loop(0, _CH // 128, unroll=4)
                    def block(bcl):
                        rp = rows_p.at[pl.ds(bcl * 128 * _P, 128 * _P)]
                        tv = t_v.at[b, pl.ds(bcl * 1024, _HALF + 1024)]
                        for d in range(16):
                            toff = (d // 8) * _HALF + (d % 8) * 128
                            for bi in range(8):
                                v = plsc.load_gather(
                                    rp, [pos17[d] + bi * 16 * _P])
                                tv[pl.ds(toff + bi * 16, 16)] = v

                    for dh in range(2):
                        pltpu.async_copy(
                            t_v.at[b, pl.ds(dh * _HALF, _HALF)],
                            out_hbm.at[h, dh, pl.ds(bcb * _HALF, _HALF)],
                            tsems[b])
            return 0

        lax.fori_loop(0, _PER_W, unit, 0)
        for b in range(2):
            for dh in range(2):
                pltpu.make_async_copy(
                    t_v.at[b, pl.ds(dh * _HALF, _HALF)],
                    out_hbm.at[0, dh, pl.ds(0, _HALF)], tsems[b]).wait()

    return k


def kernel(x, W):
    x_t = jnp.swapaxes(x, 0, 1).astype(jnp.int32)
    out5 = _make_gather()(x_t, W)
    t = out5.reshape(_H, 2, _B // 128, 8, 128)
    t = t.transpose(0, 1, 3, 2, 4)
    t = t.reshape(_H, _D, _B)
    return t.transpose(2, 0, 1)
